# baseline (speedup 1.0000x reference)
import functools

import jax
import jax.numpy as jnp
from jax import lax
from jax.experimental import pallas as pl
from jax.experimental.pallas import tpu as pltpu
from jax.experimental.pallas import tpu_sc as plsc

_B, _S, _D = 4, 4096, 768
_DFF = 3072
_EPS = 1e-05
_K = _S // 2

_RB = 2048
_TB = 256

_NC, _NS = 2, 16
_SPT = 2 * _K // _NS
_GC = 64
_CPT = 2 * _S // _NS


def _wvo_body(wv_ref, wo_ref, out_ref):
    out_ref[...] = jnp.dot(wv_ref[...], wo_ref[...],
                           preferred_element_type=jnp.float32)


def _router_body(x_ref, rw_ref, rb_ref, w_ref):
    logits = jnp.dot(x_ref[...], rw_ref[...],
                     preferred_element_type=jnp.float32)
    w_ref[...] = jax.nn.sigmoid(logits + rb_ref[0])


def _thr_body(w_ref, thr_ref):
    bits = lax.bitcast_convert_type(w_ref[...], jnp.int32)

    def body(_, carry):
        lo, hi = carry
        mid = lo + (hi - lo) // 2
        cnt = jnp.sum((bits >= mid).astype(jnp.int32), axis=1, keepdims=True)
        ge = cnt >= _K
        return jnp.where(ge, mid, lo), jnp.where(ge, hi, mid)

    lo0 = jnp.zeros((_B, 1), jnp.int32)
    hi0 = jnp.full((_B, 1), 0x7F800000, jnp.int32)
    lo, _ = lax.fori_loop(0, 31, body, (lo0, hi0))
    thr = lax.bitcast_convert_type(lo, jnp.float32)
    thr_ref[...] = jnp.broadcast_to(thr, (_B, 128))


def _sel_body(x_ref, wvo_ref, g1_ref, g2_ref, wg_ref, wu_ref, wd_ref, out_ref):
    x = x_ref[...]
    n1 = x * lax.rsqrt(jnp.mean(x * x, axis=-1, keepdims=True) + _EPS)
    n1 = n1 * g1_ref[...]
    attn = jnp.dot(n1, wvo_ref[...], preferred_element_type=jnp.float32)
    resid = x + attn
    n2 = resid * lax.rsqrt(
        jnp.mean(resid * resid, axis=-1, keepdims=True) + _EPS)
    n2 = n2 * g2_ref[...]
    a = jnp.dot(n2, wg_ref[...], preferred_element_type=jnp.float32)
    b = jnp.dot(n2, wu_ref[...], preferred_element_type=jnp.float32)
    h = jax.nn.silu(a) * b
    ffn = jnp.dot(h, wd_ref[...], preferred_element_type=jnp.float32)
    out_ref[...] = resid + ffn


def _sc_gather_body(w_hbm, thr_hbm, x_hbm, idx_hbm, xg_hbm,
                    w_v, thr_v, cidx_v, sidx_v, ring_v, sem):
    c = lax.axis_index("c")
    s = lax.axis_index("s")

    b = 2 * c + (s % 2)
    pltpu.sync_copy(w_hbm.at[b], w_v)
    pltpu.sync_copy(thr_hbm.at[pl.ds(b * 16, 16)], thr_v)
    thrv = thr_v[...]
    base = b * _S

    def body(i, off):
        wv = w_v[pl.ds(i * 16, 16)]
        m = wv >= thrv
        mi = m.astype(jnp.int32)
        pos = plsc.cumsum(mi)
        ids = lax.iota(jnp.int32, 16) + (base + i * 16)
        tgt = jnp.where(m, off + pos - 1, _S + 8)
        plsc.store_scatter(cidx_v, [tgt], ids)
        return off + jnp.sum(mi, axis=0)

    lax.fori_loop(0, _S // 16, body, jnp.int32(0))

    @pl.when(s < 2)
    def _publish():
        pltpu.sync_copy(cidx_v.at[pl.ds(0, _K)],
                        idx_hbm.at[pl.ds(b * _K, _K)])

    plsc.subcore_barrier()

    slot0 = c * (2 * _K) + s * _SPT
    for g in range(_SPT // _GC):
        pltpu.sync_copy(idx_hbm.at[pl.ds(slot0 + g * _GC, _GC)],
                        sidx_v.at[g])
    for g in range(_SPT // _GC):
        buf = ring_v.at[g % 2]
        pltpu.async_copy(x_hbm.at[sidx_v.at[g]], buf, sem).wait()
        pltpu.sync_copy(buf, xg_hbm.at[pl.ds(slot0 + g * _GC, _GC)])


def _sc_merge_body(x_hbm, comp_hbm, idx_hbm, out_hbm,
                   sidx_v, ring_v, sem, sem2):
    c = lax.axis_index("c")
    s = lax.axis_index("s")

    r0 = c * (2 * _S) + s * _CPT
    nchunks = _CPT // _GC
    cp_in = pltpu.async_copy(x_hbm.at[pl.ds(r0, _GC)], ring_v.at[0], sem)
    cp_out = None
    for g in range(nchunks):
        cp_in.wait()
        if cp_out is not None:
            cp_out.wait()
        if g + 1 < nchunks:
            cp_in = pltpu.async_copy(
                x_hbm.at[pl.ds(r0 + (g + 1) * _GC, _GC)],
                ring_v.at[(g + 1) % 2], sem)
        cp_out = pltpu.async_copy(
            ring_v.at[g % 2], out_hbm.at[pl.ds(r0 + g * _GC, _GC)], sem2)
    cp_out.wait()

    plsc.subcore_barrier()

    slot0 = c * (2 * _K) + s * _SPT
    for g in range(_SPT // _GC):
        pltpu.sync_copy(idx_hbm.at[pl.ds(slot0 + g * _GC, _GC)],
                        sidx_v.at[g])
    for g in range(_SPT // _GC):
        buf = ring_v.at[g % 2]
        pltpu.sync_copy(comp_hbm.at[pl.ds(slot0 + g * _GC, _GC)], buf)
        pltpu.async_copy(buf, out_hbm.at[sidx_v.at[g]], sem).wait()


_sc_mesh = plsc.VectorSubcoreMesh(core_axis_name="c", subcore_axis_name="s")

_sc_gather = functools.partial(
    pl.kernel,
    out_type=(jax.ShapeDtypeStruct((_B * _K,), jnp.int32),
              jax.ShapeDtypeStruct((_B * _K, _D), jnp.float32)),
    mesh=_sc_mesh,
    compiler_params=pltpu.CompilerParams(needs_layout_passes=False),
    scratch_types=[
        pltpu.VMEM((_S,), jnp.float32),
        pltpu.VMEM((16,), jnp.float32),
        pltpu.VMEM((_S + 32,), jnp.int32),
        pltpu.VMEM((_SPT // _GC, _GC), jnp.int32),
        pltpu.VMEM((2, _GC, _D), jnp.float32),
        pltpu.SemaphoreType.DMA,
    ],
)(_sc_gather_body)

_sc_merge = functools.partial(
    pl.kernel,
    out_type=jax.ShapeDtypeStruct((_B * _S, _D), jnp.float32),
    mesh=_sc_mesh,
    scratch_types=[
        pltpu.VMEM((_SPT // _GC, _GC), jnp.int32),
        pltpu.VMEM((2, _GC, _D), jnp.float32),
        pltpu.SemaphoreType.DMA,
        pltpu.SemaphoreType.DMA,
    ],
)(_sc_merge_body)


def kernel(hidden_states, router_w, router_b, wq, wk, wv, wo, g1, g2, wg, wu, wd):
    del wq, wk
    x = hidden_states

    wvo = pl.pallas_call(
        _wvo_body,
        out_shape=jax.ShapeDtypeStruct((_D, _D), jnp.float32),
    )(wv, wo)

    xf = x.reshape(_B * _S, _D)
    weights = pl.pallas_call(
        _router_body,
        grid=(_B * _S // _RB,),
        in_specs=[
            pl.BlockSpec((_RB, _D), lambda i: (i, 0)),
            pl.BlockSpec((_D, 1), lambda i: (0, 0)),
            pl.BlockSpec(memory_space=pltpu.SMEM),
        ],
        out_specs=pl.BlockSpec((_RB, 1), lambda i: (i, 0)),
        out_shape=jax.ShapeDtypeStruct((_B * _S, 1), jnp.float32),
    )(xf, router_w, router_b)

    thr = pl.pallas_call(
        _thr_body,
        out_shape=jax.ShapeDtypeStruct((_B, 128), jnp.float32),
    )(weights.reshape(_B, _S))
    thr16 = thr[:, 0:16].reshape(_B * 16)

    gidx, xg = _sc_gather(weights.reshape(_B, _S), thr16, xf)

    computed = pl.pallas_call(
        _sel_body,
        grid=(_B * _K // _TB,),
        in_specs=[
            pl.BlockSpec((_TB, _D), lambda i: (i, 0)),
            pl.BlockSpec((_D, _D), lambda i: (0, 0)),
            pl.BlockSpec((1, _D), lambda i: (0, 0)),
            pl.BlockSpec((1, _D), lambda i: (0, 0)),
            pl.BlockSpec((_D, _DFF), lambda i: (0, 0)),
            pl.BlockSpec((_D, _DFF), lambda i: (0, 0)),
            pl.BlockSpec((_DFF, _D), lambda i: (0, 0)),
        ],
        out_specs=pl.BlockSpec((_TB, _D), lambda i: (i, 0)),
        out_shape=jax.ShapeDtypeStruct((_B * _K, _D), jnp.float32),
    )(xg, wvo, g1.reshape(1, _D), g2.reshape(1, _D), wg, wu, wd)

    outf = _sc_merge(xf, computed, gidx)
    return outf.reshape(_B, _S, _D)


# speedup vs baseline: 1.0361x; 1.0361x over previous
import functools

import jax
import jax.numpy as jnp
from jax import lax
from jax.experimental import pallas as pl
from jax.experimental.pallas import tpu as pltpu
from jax.experimental.pallas import tpu_sc as plsc

_B, _S, _D = 4, 4096, 768
_DFF = 3072
_EPS = 1e-05
_K = _S // 2

_RB = 2048
_TB = 512

_NC, _NS = 2, 16
_SPT = 2 * _K // _NS
_GC = 64
_CPT = 2 * _S // _NS


def _wvo_body(wv_ref, wo_ref, out_ref):
    out_ref[...] = jnp.dot(wv_ref[...], wo_ref[...],
                           preferred_element_type=jnp.float32)


def _router_body(x_ref, rw_ref, rb_ref, w_ref):
    logits = jnp.dot(x_ref[...], rw_ref[...],
                     preferred_element_type=jnp.float32)
    w_ref[...] = jax.nn.sigmoid(logits + rb_ref[0])


def _thr_body(w_ref, thr_ref):
    bits = lax.bitcast_convert_type(w_ref[...], jnp.int32)

    def body(_, carry):
        lo, hi = carry
        mid = lo + (hi - lo) // 2
        cnt = jnp.sum((bits >= mid).astype(jnp.int32), axis=1, keepdims=True)
        ge = cnt >= _K
        return jnp.where(ge, mid, lo), jnp.where(ge, hi, mid)

    lo0 = jnp.zeros((_B, 1), jnp.int32)
    hi0 = jnp.full((_B, 1), 0x7F800000, jnp.int32)
    lo, _ = lax.fori_loop(0, 31, body, (lo0, hi0))
    thr = lax.bitcast_convert_type(lo, jnp.float32)
    thr_ref[...] = jnp.broadcast_to(thr, (_B, 128))


def _sel_body(x_ref, wvo_ref, g1_ref, g2_ref, wg_ref, wu_ref, wd_ref, out_ref):
    x = x_ref[...]
    n1 = x * lax.rsqrt(jnp.mean(x * x, axis=-1, keepdims=True) + _EPS)
    n1 = n1 * g1_ref[...]
    attn = jnp.dot(n1, wvo_ref[...], preferred_element_type=jnp.float32)
    resid = x + attn
    n2 = resid * lax.rsqrt(
        jnp.mean(resid * resid, axis=-1, keepdims=True) + _EPS)
    n2 = n2 * g2_ref[...]
    a = jnp.dot(n2, wg_ref[...], preferred_element_type=jnp.float32)
    b = jnp.dot(n2, wu_ref[...], preferred_element_type=jnp.float32)
    h = jax.nn.silu(a) * b
    ffn = jnp.dot(h, wd_ref[...], preferred_element_type=jnp.float32)
    out_ref[...] = resid + ffn


def _sc_gather_body(w_hbm, thr_hbm, x_hbm, idx_hbm, xg_hbm,
                    w_v, thr_v, cidx_v, sidx_v, ring_v, sem):
    c = lax.axis_index("c")
    s = lax.axis_index("s")

    b = 2 * c + (s % 2)
    pltpu.sync_copy(w_hbm.at[b], w_v)
    pltpu.sync_copy(thr_hbm.at[pl.ds(b * 16, 16)], thr_v)
    thrv = thr_v[...]
    base = b * _S

    def body(i, off):
        wv = w_v[pl.ds(i * 16, 16)]
        m = wv >= thrv
        mi = m.astype(jnp.int32)
        pos = plsc.cumsum(mi)
        ids = lax.iota(jnp.int32, 16) + (base + i * 16)
        tgt = jnp.where(m, off + pos - 1, _S + 8)
        plsc.store_scatter(cidx_v, [tgt], ids)
        return off + jnp.sum(mi, axis=0)

    lax.fori_loop(0, _S // 16, body, jnp.int32(0))

    @pl.when(s < 2)
    def _publish():
        pltpu.sync_copy(cidx_v.at[pl.ds(0, _K)],
                        idx_hbm.at[pl.ds(b * _K, _K)])

    plsc.subcore_barrier()

    slot0 = c * (2 * _K) + s * _SPT
    for g in range(_SPT // _GC):
        pltpu.sync_copy(idx_hbm.at[pl.ds(slot0 + g * _GC, _GC)],
                        sidx_v.at[g])
    for g in range(_SPT // _GC):
        buf = ring_v.at[g % 2]
        pltpu.async_copy(x_hbm.at[sidx_v.at[g]], buf, sem).wait()
        pltpu.sync_copy(buf, xg_hbm.at[pl.ds(slot0 + g * _GC, _GC)])


def _sc_merge_body(x_hbm, comp_hbm, idx_hbm, out_hbm,
                   sidx_v, ring_v, sem, sem2):
    c = lax.axis_index("c")
    s = lax.axis_index("s")

    r0 = c * (2 * _S) + s * _CPT
    nchunks = _CPT // _GC
    cp_in = pltpu.async_copy(x_hbm.at[pl.ds(r0, _GC)], ring_v.at[0], sem)
    cp_out = None
    for g in range(nchunks):
        cp_in.wait()
        if cp_out is not None:
            cp_out.wait()
        if g + 1 < nchunks:
            cp_in = pltpu.async_copy(
                x_hbm.at[pl.ds(r0 + (g + 1) * _GC, _GC)],
                ring_v.at[(g + 1) % 2], sem)
        cp_out = pltpu.async_copy(
            ring_v.at[g % 2], out_hbm.at[pl.ds(r0 + g * _GC, _GC)], sem2)
    cp_out.wait()

    plsc.subcore_barrier()

    slot0 = c * (2 * _K) + s * _SPT
    for g in range(_SPT // _GC):
        pltpu.sync_copy(idx_hbm.at[pl.ds(slot0 + g * _GC, _GC)],
                        sidx_v.at[g])
    for g in range(_SPT // _GC):
        buf = ring_v.at[g % 2]
        pltpu.sync_copy(comp_hbm.at[pl.ds(slot0 + g * _GC, _GC)], buf)
        pltpu.async_copy(buf, out_hbm.at[sidx_v.at[g]], sem).wait()


_sc_mesh = plsc.VectorSubcoreMesh(core_axis_name="c", subcore_axis_name="s")

_sc_gather = functools.partial(
    pl.kernel,
    out_type=(jax.ShapeDtypeStruct((_B * _K,), jnp.int32),
              jax.ShapeDtypeStruct((_B * _K, _D), jnp.float32)),
    mesh=_sc_mesh,
    compiler_params=pltpu.CompilerParams(needs_layout_passes=False),
    scratch_types=[
        pltpu.VMEM((_S,), jnp.float32),
        pltpu.VMEM((16,), jnp.float32),
        pltpu.VMEM((_S + 32,), jnp.int32),
        pltpu.VMEM((_SPT // _GC, _GC), jnp.int32),
        pltpu.VMEM((2, _GC, _D), jnp.float32),
        pltpu.SemaphoreType.DMA,
    ],
)(_sc_gather_body)

_sc_merge = functools.partial(
    pl.kernel,
    out_type=jax.ShapeDtypeStruct((_B * _S, _D), jnp.float32),
    mesh=_sc_mesh,
    scratch_types=[
        pltpu.VMEM((_SPT // _GC, _GC), jnp.int32),
        pltpu.VMEM((2, _GC, _D), jnp.float32),
        pltpu.SemaphoreType.DMA,
        pltpu.SemaphoreType.DMA,
    ],
)(_sc_merge_body)


def kernel(hidden_states, router_w, router_b, wq, wk, wv, wo, g1, g2, wg, wu, wd):
    del wq, wk
    x = hidden_states

    wvo = pl.pallas_call(
        _wvo_body,
        out_shape=jax.ShapeDtypeStruct((_D, _D), jnp.float32),
    )(wv, wo)

    xf = x.reshape(_B * _S, _D)
    weights = pl.pallas_call(
        _router_body,
        grid=(_B * _S // _RB,),
        in_specs=[
            pl.BlockSpec((_RB, _D), lambda i: (i, 0)),
            pl.BlockSpec((_D, 1), lambda i: (0, 0)),
            pl.BlockSpec(memory_space=pltpu.SMEM),
        ],
        out_specs=pl.BlockSpec((_RB, 1), lambda i: (i, 0)),
        out_shape=jax.ShapeDtypeStruct((_B * _S, 1), jnp.float32),
    )(xf, router_w, router_b)

    thr = pl.pallas_call(
        _thr_body,
        out_shape=jax.ShapeDtypeStruct((_B, 128), jnp.float32),
    )(weights.reshape(_B, _S))
    thr16 = thr[:, 0:16].reshape(_B * 16)

    gidx, xg = _sc_gather(weights.reshape(_B, _S), thr16, xf)

    computed = pl.pallas_call(
        _sel_body,
        grid=(_B * _K // _TB,),
        in_specs=[
            pl.BlockSpec((_TB, _D), lambda i: (i, 0)),
            pl.BlockSpec((_D, _D), lambda i: (0, 0)),
            pl.BlockSpec((1, _D), lambda i: (0, 0)),
            pl.BlockSpec((1, _D), lambda i: (0, 0)),
            pl.BlockSpec((_D, _DFF), lambda i: (0, 0)),
            pl.BlockSpec((_D, _DFF), lambda i: (0, 0)),
            pl.BlockSpec((_DFF, _D), lambda i: (0, 0)),
        ],
        out_specs=pl.BlockSpec((_TB, _D), lambda i: (i, 0)),
        out_shape=jax.ShapeDtypeStruct((_B * _K, _D), jnp.float32),
    )(xg, wvo, g1.reshape(1, _D), g2.reshape(1, _D), wg, wu, wd)

    outf = _sc_merge(xf, computed, gidx)
    return outf.reshape(_B, _S, _D)
